# baseline (speedup 1.0000x reference)
import jax
import jax.numpy as jnp
from jax.experimental import pallas as pl
from jax.experimental.pallas import tpu as pltpu

B_BLK = 32


def _body(x_ref, t_ref, o_ref):
    t = t_ref[...]
    norms = jnp.sqrt(jnp.sum(t * t, axis=-1, keepdims=True))
    scale = jnp.where(norms > 1.0, 1.0 / (norms + 1e-7), 1.0)
    o_ref[...] = x_ref[...] + t * scale


def kernel(x, table):
    B, S, one, D = x.shape
    x3 = x.reshape(B, S, D)
    grid = (B // B_BLK,)
    out = pl.pallas_call(
        _body,
        grid=grid,
        in_specs=[
            pl.BlockSpec((B_BLK, S, D), lambda i: (i, 0, 0)),
            pl.BlockSpec((S, D), lambda i: (0, 0)),
        ],
        out_specs=pl.BlockSpec((B_BLK, S, D), lambda i: (i, 0, 0)),
        out_shape=jax.ShapeDtypeStruct((B, S, D), x.dtype),
        compiler_params=pltpu.CompilerParams(
            dimension_semantics=("arbitrary",),
        ),
    )(x3, table)
    return out.reshape(B, S, one, D)


# speedup vs baseline: 1.0044x; 1.0044x over previous
import jax
import jax.numpy as jnp
from jax.experimental import pallas as pl
from jax.experimental.pallas import tpu as pltpu

B_BLK = 32


def _body(x_ref, t_ref, o_ref, enc_ref):
    @pl.when(pl.program_id(0) == 0)
    def _():
        t = t_ref[...]
        norms = jnp.sqrt(jnp.sum(t * t, axis=-1, keepdims=True))
        scale = jnp.where(norms > 1.0, 1.0 / (norms + 1e-7), 1.0)
        enc_ref[...] = t * scale

    o_ref[...] = x_ref[...] + enc_ref[...]


def kernel(x, table):
    B, S, one, D = x.shape
    x3 = x.reshape(B, S, D)
    grid = (B // B_BLK,)
    out = pl.pallas_call(
        _body,
        grid=grid,
        in_specs=[
            pl.BlockSpec((B_BLK, S, D), lambda i: (i, 0, 0)),
            pl.BlockSpec((S, D), lambda i: (0, 0)),
        ],
        out_specs=pl.BlockSpec((B_BLK, S, D), lambda i: (i, 0, 0)),
        out_shape=jax.ShapeDtypeStruct((B, S, D), x.dtype),
        scratch_shapes=[pltpu.VMEM((S, D), jnp.float32)],
        compiler_params=pltpu.CompilerParams(
            dimension_semantics=("arbitrary",),
        ),
    )(x3, table)
    return out.reshape(B, S, one, D)


# speedup vs baseline: 1.0349x; 1.0304x over previous
import jax
import jax.numpy as jnp
from jax.experimental import pallas as pl
from jax.experimental.pallas import tpu as pltpu

CHUNK = 32
N_SLOTS = 6


def _body(x_hbm, t_ref, o_hbm, in_buf, out_buf, in_sems, out_sems):
    n_chunks = x_hbm.shape[0] // CHUNK

    t = t_ref[...]
    norms = jnp.sqrt(jnp.sum(t * t, axis=-1, keepdims=True))
    scale = jnp.where(norms > 1.0, 1.0 / (norms + 1e-7), 1.0)
    enc = t * scale

    def fetch(i, slot):
        return pltpu.make_async_copy(
            x_hbm.at[pl.ds(i * CHUNK, CHUNK)], in_buf.at[slot], in_sems.at[slot]
        )

    def flush(i, slot):
        return pltpu.make_async_copy(
            out_buf.at[slot], o_hbm.at[pl.ds(i * CHUNK, CHUNK)], out_sems.at[slot]
        )

    for i in range(min(N_SLOTS, n_chunks)):
        fetch(i, i).start()

    for i in range(n_chunks):
        slot = i % N_SLOTS
        fetch(i, slot).wait()
        if i >= N_SLOTS:
            flush(i - N_SLOTS, slot).wait()
        out_buf[slot] = in_buf[slot] + enc
        flush(i, slot).start()
        nxt = i + N_SLOTS
        if nxt < n_chunks:
            fetch(nxt, slot).start()

    for i in range(max(0, n_chunks - N_SLOTS), n_chunks):
        flush(i, i % N_SLOTS).wait()


def kernel(x, table):
    B, S, one, D = x.shape
    x3 = x.reshape(B, S, D)
    out = pl.pallas_call(
        _body,
        in_specs=[
            pl.BlockSpec(memory_space=pltpu.HBM),
            pl.BlockSpec(memory_space=pltpu.VMEM),
        ],
        out_specs=pl.BlockSpec(memory_space=pltpu.HBM),
        out_shape=jax.ShapeDtypeStruct((B, S, D), x.dtype),
        scratch_shapes=[
            pltpu.VMEM((N_SLOTS, CHUNK, S, D), jnp.float32),
            pltpu.VMEM((N_SLOTS, CHUNK, S, D), jnp.float32),
            pltpu.SemaphoreType.DMA((N_SLOTS,)),
            pltpu.SemaphoreType.DMA((N_SLOTS,)),
        ],
    )(x3, table)
    return out.reshape(B, S, one, D)


# speedup vs baseline: 1.0405x; 1.0054x over previous
import jax
import jax.numpy as jnp
from jax.experimental import pallas as pl
from jax.experimental.pallas import tpu as pltpu

CHUNK = 128
N_SLOTS = 4
PREFETCH = 2


def _body(x_hbm, t_ref, o_hbm, buf, in_sems, out_sems):
    n = x_hbm.shape[0] // CHUNK

    t = t_ref[...]
    norms = jnp.sqrt(jnp.sum(t * t, axis=-1, keepdims=True))
    scale = jnp.where(norms > 1.0, 1.0 / (norms + 1e-7), 1.0)
    enc = t * scale

    def fetch(i):
        slot = i % N_SLOTS
        return pltpu.make_async_copy(
            x_hbm.at[pl.ds(i * CHUNK, CHUNK)], buf.at[slot], in_sems.at[slot]
        )

    def flush(i):
        slot = i % N_SLOTS
        return pltpu.make_async_copy(
            buf.at[slot], o_hbm.at[pl.ds(i * CHUNK, CHUNK)], out_sems.at[slot]
        )

    for i in range(min(PREFETCH, n)):
        fetch(i).start()

    for i in range(n):
        slot = i % N_SLOTS
        fetch(i).wait()
        nxt = i + PREFETCH
        if nxt < n:
            if nxt - N_SLOTS >= 0:
                flush(nxt - N_SLOTS).wait()
            fetch(nxt).start()
        buf[slot] = buf[slot] + enc
        flush(i).start()

    for i in range(max(0, n - N_SLOTS), n):
        flush(i).wait()


def kernel(x, table):
    B, S, one, D = x.shape
    x3 = x.reshape(B, S, D)
    out = pl.pallas_call(
        _body,
        in_specs=[
            pl.BlockSpec(memory_space=pltpu.HBM),
            pl.BlockSpec(memory_space=pltpu.VMEM),
        ],
        out_specs=pl.BlockSpec(memory_space=pltpu.HBM),
        out_shape=jax.ShapeDtypeStruct((B, S, D), x.dtype),
        scratch_shapes=[
            pltpu.VMEM((N_SLOTS, CHUNK, S, D), jnp.float32),
            pltpu.SemaphoreType.DMA((N_SLOTS,)),
            pltpu.SemaphoreType.DMA((N_SLOTS,)),
        ],
    )(x3, table)
    return out.reshape(B, S, one, D)
